# baseline (speedup 1.0000x reference)
import functools

import jax
import jax.numpy as jnp
from jax import lax
from jax.experimental import pallas as pl
from jax.experimental.pallas import tpu as pltpu
from jax.experimental.pallas import tpu_sc as plsc

_NUM_WORKERS = 32
_LANES = 16


@functools.lru_cache(maxsize=None)
def _make_sc_gather(batch: int, rdim: int, cols: int, nidx: int):
    assert (batch, rdim, cols, nidx) == (4, 4096, 2048, 2), "offset math is shape-specialized"
    total = batch * rdim * nidx
    per_w = total // _NUM_WORKERS
    pr = per_w // 128
    steps = per_w // _LANES
    row_stride = 8 * cols

    mesh = plsc.VectorSubcoreMesh(core_axis_name="c", subcore_axis_name="s")

    @functools.partial(
        pl.kernel,
        out_type=jax.ShapeDtypeStruct((_NUM_WORKERS, pr, 128), jnp.float32),
        mesh=mesh,
        scratch_types=[
            pltpu.VMEM((_LANES,), jnp.int32),
            pltpu.VMEM((pr, 128), jnp.int32),
            pltpu.VMEM((pr, 128), jnp.float32),
            pltpu.SemaphoreType.DMA,
            pltpu.SemaphoreType.DMA,
            pltpu.SemaphoreType.DMA,
        ],
    )
    def sc_gather(inp_hbm, idx_hbm, out_hbm, pat_v, idx_v, gat_v, sem_a, sem_b, sem_o):
        wid = lax.axis_index("s") * 2 + lax.axis_index("c")
        pltpu.sync_copy(idx_hbm, pat_v.at[pl.ds(0, nidx)])
        v = pat_v[...]
        cphys = ((v >> 7) << 10) + (v & 127)
        lane = lax.iota(jnp.int32, _LANES)
        lane_vec = jnp.right_shift(lane, 3) * row_stride + (lane & 7) * 128
        w_base = ((wid >> 3) * (rdim // 8) + (wid & 7) * 64) * row_stride
        dnums = lax.GatherDimensionNumbers(
            offset_dims=(), collapsed_slice_dims=(0,), start_index_map=(0,)
        )
        base = [
            lax.gather(
                cphys,
                jnp.full((_LANES, 1), j, jnp.int32),
                dnums,
                slice_sizes=(1,),
                mode=lax.GatherScatterMode.PROMISE_IN_BOUNDS,
            )
            + lane_vec + w_base
            for j in range(nidx)
        ]
        half = steps // 32

        def make_body(sem):
            def body(rt_lo, carry):
                for j in range(nidx):
                    r = rt_lo * nidx + j
                    roff = rt_lo * (16 * row_stride)
                    for k in range(8):
                        idx_v[r, pl.ds(k * _LANES, _LANES)] = (
                            base[j] + roff + (k * 2) * row_stride
                        )
                    pltpu.async_copy(inp_hbm.at[idx_v.at[r]], gat_v.at[r], sem)
                return carry
            return body

        lax.fori_loop(0, half, make_body(sem_a), 0)
        lax.fori_loop(half, 2 * half, make_body(sem_b), 0)
        hr = pr // 2
        out_copies = []
        for g, sem in enumerate((sem_a, sem_b)):
            for r in range(hr):
                pltpu.make_async_copy(
                    inp_hbm.at[pl.ds(0, 128)], gat_v.at[g * hr + r], sem
                ).wait()
            out_copies.append(
                pltpu.async_copy(
                    gat_v.at[pl.ds(g * hr, hr)],
                    out_hbm.at[wid, pl.ds(g * hr, hr)],
                    sem_o,
                )
            )
        for c in out_copies:
            c.wait()

    return sc_gather


def kernel(input, indices):
    b, r, cols = input.shape
    (nidx,) = indices.shape
    rows = b * r
    x = input.reshape(rows // 8, 8, cols // 128, 128).transpose(0, 2, 1, 3)
    out = _make_sc_gather(b, r, cols, nidx)(
        x.reshape(-1), indices.astype(jnp.int32)
    )
    o4 = out.reshape(b, r // 128, nidx, 128)
    return o4.transpose(0, 1, 3, 2).reshape(b, r, nidx)
